# baseline (speedup 1.0000x reference)
import functools

import jax
import jax.numpy as jnp
from jax import lax
from jax.experimental import pallas as pl
from jax.experimental.pallas import tpu as pltpu
from jax.experimental.pallas import tpu_sc as plsc

BB, NN = 2, 1000
PH, PW = 7, 7
CH = 256
NBOX = BB * NN
ROWS = NBOX * PH * PW
NWORK = 32
ROWS_PER_W = 3136
CHUNK = 16
NCHUNK = ROWS_PER_W // CHUNK
_BASES = tuple(min(w * ROWS_PER_W, ROWS - ROWS_PER_W) for w in range(NWORK))

_HL = (256, 128, 64, 32)
_OFF = (0,
        BB * 256 * 256,
        BB * 256 * 256 + BB * 128 * 128,
        BB * 256 * 256 + BB * 128 * 128 + BB * 64 * 64)
_TROWS = _OFF[3] + BB * 32 * 32


def _prep_body(boxes_ref, meta_ref, idx_ref, w_ref):
    b4 = boxes_ref[:]
    y1 = b4[:, 0:1]
    x1 = b4[:, 1:2]
    y2 = b4[:, 2:3]
    x2 = b4[:, 3:4]
    h = y2 - y1
    w = x2 - x1
    ih = meta_ref[0:1, 4:5]
    iw = meta_ref[0:1, 5:6]
    area = ih * iw
    lvl_f = jnp.log(jnp.sqrt(h * w) / (224.0 / jnp.sqrt(area))) / jnp.log(2.0)
    lvl = jnp.minimum(5, jnp.maximum(2, 4 + jnp.round(lvl_f).astype(jnp.int32)))

    hf = jnp.where(lvl == 2, 256.0,
                   jnp.where(lvl == 3, 128.0,
                             jnp.where(lvl == 4, 64.0, 32.0)))
    hi = hf.astype(jnp.int32)
    hw_rows = hi * hi
    off = jnp.where(lvl == 2, _OFF[0],
                    jnp.where(lvl == 3, _OFF[1],
                              jnp.where(lvl == 4, _OFF[2], _OFF[3])))
    bidx = (lax.broadcasted_iota(jnp.int32, (NBOX, 1), 0) >= NN).astype(jnp.int32)

    q = lax.broadcasted_iota(jnp.int32, (NBOX, PH * PW * 8), 1)
    pi = q // (PW * 8)
    pj = (q // 8) % PW
    c = (q // 2) % 4
    hh = q % 2
    fy = pi.astype(jnp.float32)
    fx = pj.astype(jnp.float32)
    ys = y1 * (hf - 1.0) + fy * (h * (hf - 1.0) / 6.0)
    xs = x1 * (hf - 1.0) + fx * (w * (hf - 1.0) / 6.0)
    him1 = hi - 1
    yy0 = jnp.clip(jnp.floor(ys).astype(jnp.int32), 0, him1)
    yy1 = jnp.minimum(yy0 + 1, him1)
    xx0 = jnp.clip(jnp.floor(xs).astype(jnp.int32), 0, him1)
    xx1 = jnp.minimum(xx0 + 1, him1)
    cy = jnp.where(c >= 2, yy1, yy0)
    cx = jnp.where(c % 2 == 1, xx1, xx0)
    idx_ref[:] = (off + bidx * hw_rows + cy * hi + cx) * 2 + hh

    q2 = lax.broadcasted_iota(jnp.int32, (NBOX, PH * PW * 4), 1)
    pi2 = q2 // (PW * 4)
    pj2 = (q2 // 4) % PW
    c2 = q2 % 4
    ys2 = y1 * (hf - 1.0) + pi2.astype(jnp.float32) * (h * (hf - 1.0) / 6.0)
    xs2 = x1 * (hf - 1.0) + pj2.astype(jnp.float32) * (w * (hf - 1.0) / 6.0)
    wy = ys2 - jnp.floor(ys2)
    wx = xs2 - jnp.floor(xs2)
    wyc = jnp.where(c2 >= 2, wy, 1.0 - wy)
    wxc = jnp.where(c2 % 2 == 1, wx, 1.0 - wx)
    wgt = wyc * wxc
    valid = ((ys2 >= 0.0) & (ys2 <= hf - 1.0) & (xs2 >= 0.0)
             & (xs2 <= hf - 1.0))
    w_ref[:] = jnp.where(valid, wgt, 0.0)


_prep = pl.pallas_call(
    _prep_body,
    out_shape=[
        jax.ShapeDtypeStruct((NBOX, PH * PW * 8), jnp.int32),
        jax.ShapeDtypeStruct((NBOX, PH * PW * 4), jnp.float32),
    ],
)


@functools.cache
def _make_sc_pool():
    @functools.partial(
        pl.kernel,
        out_type=jax.ShapeDtypeStruct((ROWS * 2, 128), jnp.float32),
        mesh=plsc.VectorSubcoreMesh(core_axis_name="c", subcore_axis_name="s"),
        scratch_types=[
            pltpu.VMEM((NCHUNK * CHUNK * 8,), jnp.int32),
            pltpu.VMEM((NCHUNK * CHUNK * 4,), jnp.float32),
            pltpu.VMEM((CHUNK * 8, 128), jnp.float32),
            pltpu.VMEM((CHUNK * 8, 128), jnp.float32),
            pltpu.VMEM((CHUNK * 2, 128), jnp.float32),
            pltpu.VMEM((CHUNK * 2, 128), jnp.float32),
            pltpu.SemaphoreType.DMA,
            pltpu.SemaphoreType.DMA,
            pltpu.SemaphoreType.DMA,
            pltpu.SemaphoreType.DMA,
        ],
    )
    def _sc_pool(table_hbm, idx_hbm, w_hbm, out_hbm, idx_all, w_all,
                 rows_a, rows_b, out_a, out_b, gs_a, gs_b, os_a, os_b):
        wid = lax.axis_index("s") * 2 + lax.axis_index("c")
        base_w = jnp.where(wid == NWORK - 1, ROWS - ROWS_PER_W,
                           wid * ROWS_PER_W)
        rows_bufs = (rows_a, rows_b)
        out_bufs = (out_a, out_b)
        g_sems = (gs_a, gs_b)
        o_sems = (os_a, os_b)
        ipw = NCHUNK * CHUNK * 8
        wpw = NCHUNK * CHUNK * 4

        pltpu.sync_copy(idx_hbm.at[pl.ds(wid * ipw, ipw)], idx_all)
        pltpu.sync_copy(w_hbm.at[pl.ds(wid * wpw, wpw)], w_all)

        def start_gather(ci, b):
            pltpu.async_copy(
                table_hbm.at[idx_all.at[pl.ds(ci * CHUNK * 8, CHUNK * 8)]],
                rows_bufs[b], g_sems[b])

        def compute(ci, b):
            rows_v = rows_bufs[b]
            out_v = out_bufs[b]

            def grp_body(g, gcarry):
                w16 = w_all[pl.ds(ci * CHUNK * 4 + 16 * g, 16)]
                for rr in range(4):
                    r = 4 * g + rr
                    w0 = w16[4 * rr]
                    w1 = w16[4 * rr + 1]
                    w2 = w16[4 * rr + 2]
                    w3 = w16[4 * rr + 3]
                    for hh in range(2):
                        for jv in range(8):
                            s = pl.ds(jv * 16, 16)
                            acc = (rows_v[8 * r + hh, s] * w0
                                   + rows_v[8 * r + 2 + hh, s] * w1
                                   + rows_v[8 * r + 4 + hh, s] * w2
                                   + rows_v[8 * r + 6 + hh, s] * w3)
                            out_v[2 * r + hh, s] = acc
                return gcarry

            lax.fori_loop(0, CHUNK // 4, grp_body, 0)

        start_gather(0, 0)

        def pair_body(ci0, carry):
            for b in range(2):
                ci = 2 * ci0 + b
                nb = 1 - b

                @pl.when(ci + 1 < NCHUNK)
                def _():
                    start_gather(ci + 1, nb)

                pltpu.make_async_copy(
                    table_hbm.at[idx_all.at[pl.ds(ci * CHUNK * 8, CHUNK * 8)]],
                    rows_bufs[b], g_sems[b]).wait()

                @pl.when(ci >= 2)
                def _():
                    pltpu.make_async_copy(
                        out_bufs[b], out_hbm.at[pl.ds(0, CHUNK * 2)],
                        o_sems[b]).wait()

                compute(ci, b)
                obase = (base_w + ci * CHUNK) * 2
                pltpu.async_copy(out_bufs[b],
                                 out_hbm.at[pl.ds(obase, CHUNK * 2)],
                                 o_sems[b])
            return carry

        lax.fori_loop(0, NCHUNK // 2, pair_body, 0)
        for b in range(2):
            pltpu.make_async_copy(out_bufs[b], out_hbm.at[pl.ds(0, CHUNK * 2)],
                                  o_sems[b]).wait()

    return _sc_pool


def kernel(boxes, image_meta, feature_map_p2, feature_map_p3, feature_map_p4,
           feature_map_p5):
    table = jnp.concatenate([
        feature_map_p2.reshape(-1, 128),
        feature_map_p3.reshape(-1, 128),
        feature_map_p4.reshape(-1, 128),
        feature_map_p5.reshape(-1, 128),
    ], axis=0)
    idx, wgt = _prep(boxes.reshape(NBOX, 4), image_meta)
    idx_flat = idx.reshape(-1)
    w_flat = wgt.reshape(-1)
    idx_slabs = jnp.concatenate(
        [idx_flat[b * 8:(b + ROWS_PER_W) * 8] for b in _BASES])
    w_slabs = jnp.concatenate(
        [w_flat[b * 4:(b + ROWS_PER_W) * 4] for b in _BASES])
    out = _make_sc_pool()(table, idx_slabs, w_slabs)
    return out.reshape(BB, NN, PH, PW, CH)


# speedup vs baseline: 2.0462x; 2.0462x over previous
import functools

import jax
import jax.numpy as jnp
from jax import lax
from jax.experimental import pallas as pl
from jax.experimental.pallas import tpu as pltpu
from jax.experimental.pallas import tpu_sc as plsc

BB, NN = 2, 1000
PH, PW = 7, 7
CH = 256
NBOX = BB * NN
ROWS = NBOX * PH * PW
NWORK = 32
ROWS_PER_W = 3136
CHUNK = 32
NCHUNK = ROWS_PER_W // CHUNK
_BASES = tuple(min(w * ROWS_PER_W, ROWS - ROWS_PER_W) for w in range(NWORK))

_HL = (256, 128, 64, 32)
_OFF = (0,
        BB * 256 * 256,
        BB * 256 * 256 + BB * 128 * 128,
        BB * 256 * 256 + BB * 128 * 128 + BB * 64 * 64)
_TROWS = _OFF[3] + BB * 32 * 32


def _prep_body(boxes_ref, meta_ref, idx_ref, w_ref, oidx_ref):
    b4 = boxes_ref[:]
    y1 = b4[:, 0:1]
    x1 = b4[:, 1:2]
    y2 = b4[:, 2:3]
    x2 = b4[:, 3:4]
    h = y2 - y1
    w = x2 - x1
    ih = meta_ref[0:1, 4:5]
    iw = meta_ref[0:1, 5:6]
    area = ih * iw
    lvl_f = jnp.log(jnp.sqrt(h * w) / (224.0 / jnp.sqrt(area))) / jnp.log(2.0)
    lvl = jnp.minimum(5, jnp.maximum(2, 4 + jnp.round(lvl_f).astype(jnp.int32)))

    hf = jnp.where(lvl == 2, 256.0,
                   jnp.where(lvl == 3, 128.0,
                             jnp.where(lvl == 4, 64.0, 32.0)))
    hi = hf.astype(jnp.int32)
    hw_rows = hi * hi
    off = jnp.where(lvl == 2, _OFF[0],
                    jnp.where(lvl == 3, _OFF[1],
                              jnp.where(lvl == 4, _OFF[2], _OFF[3])))
    bidx = (lax.broadcasted_iota(jnp.int32, (NBOX, 1), 0) >= NN).astype(jnp.int32)

    q = lax.broadcasted_iota(jnp.int32, (NBOX, PH * PW * 4), 1)
    pi = q // (PW * 4)
    pj = (q // 4) % PW
    c = q % 4
    fy = pi.astype(jnp.float32)
    fx = pj.astype(jnp.float32)
    ys = y1 * (hf - 1.0) + fy * (h * (hf - 1.0) / 6.0)
    xs = x1 * (hf - 1.0) + fx * (w * (hf - 1.0) / 6.0)
    him1 = hi - 1
    yy0 = jnp.clip(jnp.floor(ys).astype(jnp.int32), 0, him1)
    yy1 = jnp.minimum(yy0 + 1, him1)
    xx0 = jnp.clip(jnp.floor(xs).astype(jnp.int32), 0, him1)
    xx1 = jnp.minimum(xx0 + 1, him1)
    cy = jnp.where(c >= 2, yy1, yy0)
    cx = jnp.where(c % 2 == 1, xx1, xx0)
    idx_ref[:] = off + bidx * hw_rows + cy * hi + cx

    q3 = lax.broadcasted_iota(jnp.int32, (NBOX, PH * PW * 2), 1)
    pix3 = q3 // 2
    h3 = q3 % 2
    i3 = pix3 // PW
    j3 = pix3 % PW
    nn = lax.broadcasted_iota(jnp.int32, (NBOX, 1), 0)
    bb3 = nn // NN
    n3 = nn - bb3 * NN
    oidx_ref[:] = (((bb3 * PH + i3) * PW + j3) * (2 * NN)
                   + (n3 // 8) * 16 + h3 * 8 + n3 % 8)

    q2 = lax.broadcasted_iota(jnp.int32, (NBOX, PH * PW * 4), 1)
    pi2 = q2 // (PW * 4)
    pj2 = (q2 // 4) % PW
    c2 = q2 % 4
    ys2 = y1 * (hf - 1.0) + pi2.astype(jnp.float32) * (h * (hf - 1.0) / 6.0)
    xs2 = x1 * (hf - 1.0) + pj2.astype(jnp.float32) * (w * (hf - 1.0) / 6.0)
    wy = ys2 - jnp.floor(ys2)
    wx = xs2 - jnp.floor(xs2)
    wyc = jnp.where(c2 >= 2, wy, 1.0 - wy)
    wxc = jnp.where(c2 % 2 == 1, wx, 1.0 - wx)
    wgt = wyc * wxc
    valid = ((ys2 >= 0.0) & (ys2 <= hf - 1.0) & (xs2 >= 0.0)
             & (xs2 <= hf - 1.0))
    w_ref[:] = jnp.where(valid, wgt, 0.0)


_prep = pl.pallas_call(
    _prep_body,
    out_shape=[
        jax.ShapeDtypeStruct((NBOX, PH * PW * 4), jnp.int32),
        jax.ShapeDtypeStruct((NBOX, PH * PW * 4), jnp.float32),
        jax.ShapeDtypeStruct((NBOX, PH * PW * 2), jnp.int32),
    ],
)


@functools.cache
def _make_sc_pool():
    @functools.partial(
        pl.kernel,
        out_type=jax.ShapeDtypeStruct((ROWS * 2, 128), jnp.float32),
        mesh=plsc.VectorSubcoreMesh(core_axis_name="c", subcore_axis_name="s"),
        scratch_types=[
            pltpu.VMEM((NCHUNK * CHUNK * 4,), jnp.int32),
            pltpu.VMEM((NCHUNK * CHUNK * 4,), jnp.float32),
            pltpu.VMEM((NCHUNK * CHUNK * 2,), jnp.int32),
            pltpu.VMEM((CHUNK * 4, CH), jnp.float32),
            pltpu.VMEM((CHUNK * 4, CH), jnp.float32),
            pltpu.VMEM((CHUNK * 2, 128), jnp.float32),
            pltpu.VMEM((CHUNK * 2, 128), jnp.float32),
            pltpu.VMEM((CHUNK * 2,), jnp.int32),
            pltpu.VMEM((CHUNK * 2,), jnp.int32),
            pltpu.SemaphoreType.DMA,
            pltpu.SemaphoreType.DMA,
            pltpu.SemaphoreType.DMA,
            pltpu.SemaphoreType.DMA,
        ],
    )
    def _sc_pool(table_hbm, idx_hbm, w_hbm, oidx_hbm, out_hbm,
                 idx_all, w_all, oidx_all, rows_a, rows_b, out_a, out_b,
                 ob_a, ob_b, gs_a, gs_b, os_a, os_b):
        wid = lax.axis_index("s") * 2 + lax.axis_index("c")
        rows_bufs = (rows_a, rows_b)
        out_bufs = (out_a, out_b)
        oidx_bufs = (ob_a, ob_b)
        g_sems = (gs_a, gs_b)
        o_sems = (os_a, os_b)
        ipw = NCHUNK * CHUNK * 4
        opw = NCHUNK * CHUNK * 2

        pltpu.sync_copy(idx_hbm.at[pl.ds(wid * ipw, ipw)], idx_all)
        pltpu.sync_copy(w_hbm.at[pl.ds(wid * ipw, ipw)], w_all)
        pltpu.sync_copy(oidx_hbm.at[pl.ds(wid * opw, opw)], oidx_all)

        def start_gather(ci, b):
            pltpu.async_copy(
                table_hbm.at[idx_all.at[pl.ds(ci * CHUNK * 4, CHUNK * 4)]],
                rows_bufs[b], g_sems[b])

        def compute(ci, b):
            rows_v = rows_bufs[b]
            out_v = out_bufs[b]

            def grp_body(g, gcarry):
                w16 = w_all[pl.ds(ci * CHUNK * 4 + 16 * g, 16)]
                for rr in range(4):
                    r = 4 * g + rr
                    w0 = w16[4 * rr]
                    w1 = w16[4 * rr + 1]
                    w2 = w16[4 * rr + 2]
                    w3 = w16[4 * rr + 3]
                    for hh in range(2):
                        for jv in range(8):
                            s = pl.ds(hh * 128 + jv * 16, 16)
                            acc = (rows_v[4 * r, s] * w0
                                   + rows_v[4 * r + 1, s] * w1
                                   + rows_v[4 * r + 2, s] * w2
                                   + rows_v[4 * r + 3, s] * w3)
                            out_v[2 * r + hh, pl.ds(jv * 16, 16)] = acc
                return gcarry

            lax.fori_loop(0, CHUNK // 4, grp_body, 0)

        start_gather(0, 0)

        def pair_body(ci0, carry):
            for b in range(2):
                ci = 2 * ci0 + b
                nb = 1 - b

                @pl.when(ci + 1 < NCHUNK)
                def _():
                    start_gather(ci + 1, nb)

                pltpu.make_async_copy(
                    table_hbm.at[idx_all.at[pl.ds(ci * CHUNK * 4, CHUNK * 4)]],
                    rows_bufs[b], g_sems[b]).wait()

                @pl.when(ci >= 2)
                def _():
                    pltpu.make_async_copy(
                        out_bufs[b], out_hbm.at[oidx_bufs[b]],
                        o_sems[b]).wait()

                for t in range(CHUNK * 2 // 16):
                    oidx_bufs[b][pl.ds(16 * t, 16)] = (
                        oidx_all[pl.ds(ci * CHUNK * 2 + 16 * t, 16)])

                compute(ci, b)
                pltpu.async_copy(out_bufs[b], out_hbm.at[oidx_bufs[b]],
                                 o_sems[b])
            return carry

        lax.fori_loop(0, NCHUNK // 2, pair_body, 0)
        for b in range(2):
            pltpu.make_async_copy(out_bufs[b], out_hbm.at[oidx_bufs[b]],
                                  o_sems[b]).wait()

    return _sc_pool


def kernel(boxes, image_meta, feature_map_p2, feature_map_p3, feature_map_p4,
           feature_map_p5):
    table = jnp.concatenate([
        feature_map_p2.reshape(-1, CH),
        feature_map_p3.reshape(-1, CH),
        feature_map_p4.reshape(-1, CH),
        feature_map_p5.reshape(-1, CH),
    ], axis=0)
    idx, wgt, oidx = _prep(boxes.reshape(NBOX, 4), image_meta)
    idx_flat = idx.reshape(-1)
    w_flat = wgt.reshape(-1)
    oidx_flat = oidx.reshape(-1)
    idx_slabs = jnp.concatenate(
        [idx_flat[b * 4:(b + ROWS_PER_W) * 4] for b in _BASES])
    w_slabs = jnp.concatenate(
        [w_flat[b * 4:(b + ROWS_PER_W) * 4] for b in _BASES])
    oidx_slabs = jnp.concatenate(
        [oidx_flat[b * 2:(b + ROWS_PER_W) * 2] for b in _BASES])
    out = _make_sc_pool()(table, idx_slabs, w_slabs, oidx_slabs)
    out7 = out.reshape(BB, PH, PW, NN // 8, 2, 8, 128)
    return out7.transpose(0, 3, 5, 1, 2, 4, 6).reshape(BB, NN, PH, PW, CH)
